# baseline (speedup 1.0000x reference)
import functools

import jax
import jax.numpy as jnp
from jax import lax
from jax.experimental import pallas as pl
from jax.experimental.pallas import tpu as pltpu
from jax.experimental.pallas import tpu_sc as plsc

NBUF = 4
LANES = 16


def _sc_geometry():
    try:
        info = plsc.get_sparse_core_info()
        return info.num_cores, info.num_subcores
    except Exception:
        return 2, 16


def _make_encoder(B, L, V, D, NC, NS):
    NW = NC * NS
    assert B % NW == 0
    BPW = B // NW
    assert BPW % NBUF == 0
    G = BPW // NBUF
    nvec = D // LANES
    inv_l = jnp.float32(1.0 / L)

    mesh = plsc.VectorSubcoreMesh(core_axis_name="c", subcore_axis_name="s")

    @functools.partial(
        pl.kernel,
        out_type=jax.ShapeDtypeStruct((B, D), jnp.float32),
        mesh=mesh,
        scratch_types=[
            pltpu.VMEM((BPW, L), jnp.int32),
            pltpu.VMEM((NBUF, L, D), jnp.float32),
            pltpu.VMEM((BPW, D), jnp.float32),
        ] + [pltpu.SemaphoreType.DMA] * NBUF,
        compiler_params=pltpu.CompilerParams(use_tc_tiling_on_sc=False),
    )
    def enc(cap_hbm, table_hbm, out_hbm, idx_v, rows_v, out_v, *sems):
        wid = lax.axis_index("s") * NC + lax.axis_index("c")
        base = wid * BPW

        pltpu.sync_copy(cap_hbm.at[pl.ds(base, BPW)], idx_v)

        def start(i, b):
            pltpu.async_copy(table_hbm.at[idx_v.at[i]], rows_v.at[b], sems[b])

        def wait(i, b):
            pltpu.make_async_copy(
                table_hbm.at[idx_v.at[i]], rows_v.at[b], sems[b]
            ).wait()

        for b in range(NBUF):
            start(jnp.int32(b), b)

        def group(g, carry):
            for b in range(NBUF):
                i = g * NBUF + b
                wait(i, b)
                accs = [jnp.zeros((LANES,), jnp.float32) for _ in range(nvec)]
                for r in range(L):
                    for v in range(nvec):
                        accs[v] = accs[v] + rows_v[b, r, pl.ds(v * LANES, LANES)]
                for v in range(nvec):
                    out_v[i, pl.ds(v * LANES, LANES)] = accs[v] * inv_l

                @pl.when(g < G - 1)
                def _():
                    start(i + NBUF, b)

            return carry

        lax.fori_loop(0, G, group, jnp.int32(0))

        pltpu.sync_copy(out_v, out_hbm.at[pl.ds(base, BPW)])

    return enc


def kernel(captions, cap_features, w2v_table):
    del cap_features
    B, L = captions.shape
    V, D = w2v_table.shape
    NC, NS = _sc_geometry()
    enc = _make_encoder(B, L, V, D, NC, NS)
    return enc(captions, w2v_table)
